# Initial kernel scaffold; baseline (speedup 1.0000x reference)
import jax
import jax.numpy as jnp
from jax.experimental import pallas as pl


def kernel(x356, x350, w1, gamma, beta, run_mean, run_var, w2):
    raise NotImplementedError("write your pallas kernel here")


# speedup vs baseline: 1.5672x; 1.5672x over previous
import jax
import jax.numpy as jnp
from jax.experimental import pallas as pl
from jax.experimental.pallas import tpu as pltpu


def _fused_bottleneck(x_ref, skip_ref, w1_ref, b_ref, w2_ref, o_ref):
    x = x_ref[...].astype(jnp.bfloat16)
    h = jnp.dot(w1_ref[...], x, preferred_element_type=jnp.float32)
    h = jnp.maximum(h + b_ref[...] + skip_ref[...], 0.0)
    o_ref[...] = jnp.dot(w2_ref[...], h.astype(jnp.bfloat16),
                         preferred_element_type=jnp.float32)


def kernel(x356, x350, w1, gamma, beta, run_mean, run_var, w2, eps=1e-5):
    N, Cin, H, W = x356.shape
    Cmid = w1.shape[0]
    Cout = w2.shape[0]
    HW = H * W

    t_hw = 256
    num_sp = pl.cdiv(HW, t_hw)

    x3 = x356.reshape(N, Cin, HW)
    s3 = x350.reshape(N, Cmid, HW)

    inv = gamma * jax.lax.rsqrt(run_var + eps)
    b = (beta - run_mean * inv)[:, None]
    w1f = (w1.reshape(Cmid, Cin) * inv[:, None]).astype(jnp.bfloat16)
    w2f = w2.reshape(Cout, Cmid).astype(jnp.bfloat16)

    flops = 2 * N * num_sp * t_hw * Cmid * (Cin + Cout)
    bytes_accessed = (x3.size + s3.size + N * Cout * HW) * 4 + (
        w1f.size + w2f.size) * 2 + b.size * 4

    out = pl.pallas_call(
        _fused_bottleneck,
        out_shape=jax.ShapeDtypeStruct((N, Cout, HW), jnp.float32),
        grid=(N, num_sp),
        in_specs=[
            pl.BlockSpec((None, Cin, t_hw), lambda n, s: (n, 0, s)),
            pl.BlockSpec((None, Cmid, t_hw), lambda n, s: (n, 0, s)),
            pl.BlockSpec((Cmid, Cin), lambda n, s: (0, 0)),
            pl.BlockSpec((Cmid, 1), lambda n, s: (0, 0)),
            pl.BlockSpec((Cout, Cmid), lambda n, s: (0, 0)),
        ],
        out_specs=pl.BlockSpec((None, Cout, t_hw), lambda n, s: (n, 0, s)),
        compiler_params=pltpu.CompilerParams(
            dimension_semantics=("parallel", "parallel"),
            vmem_limit_bytes=64 * 1024 * 1024,
        ),
        cost_estimate=pl.CostEstimate(
            flops=flops, transcendentals=0, bytes_accessed=bytes_accessed),
    )(x3, s3, w1f, b, w2f)

    return out.reshape(N, Cout, H, W)


# speedup vs baseline: 4.9259x; 3.1431x over previous
import jax
import jax.numpy as jnp
from jax.experimental import pallas as pl
from jax.experimental.pallas import tpu as pltpu


def _fused_bottleneck(x_ref, s_ref, w1_ref, b_ref, w2_ref, o_ref):
    xb = x_ref[...].astype(jnp.bfloat16)
    h = jnp.dot(xb, w1_ref[...], preferred_element_type=jnp.float32)
    h = jnp.maximum(h + b_ref[...] + s_ref[...], 0.0)
    o_ref[...] = jnp.dot(h.astype(jnp.bfloat16), w2_ref[...],
                         preferred_element_type=jnp.float32)


def kernel(x356, x350, w1, gamma, beta, run_mean, run_var, w2, eps=1e-5):
    N, Cin, H, W = x356.shape
    Cmid = w1.shape[0]
    Cout = w2.shape[0]
    M = N * H * W

    x = x356.transpose(2, 3, 0, 1).reshape(M, Cin)
    s = x350.transpose(2, 3, 0, 1).reshape(M, Cmid)

    inv = gamma * jax.lax.rsqrt(run_var + eps)
    b = (beta - run_mean * inv)[None, :]
    w1t = (w1.reshape(Cmid, Cin).T * inv[None, :]).astype(jnp.bfloat16)
    w2t = w2.reshape(Cout, Cmid).T.astype(jnp.bfloat16)

    bm = 392
    grid = (M // bm,)

    flops = 2 * M * Cmid * (Cin + Cout)
    bytes_accessed = (x.size + s.size + M * Cout) * 4 + (
        w1t.size + w2t.size) * 2 + b.size * 4

    out = pl.pallas_call(
        _fused_bottleneck,
        out_shape=jax.ShapeDtypeStruct((M, Cout), jnp.float32),
        grid=grid,
        in_specs=[
            pl.BlockSpec((bm, Cin), lambda i: (i, 0)),
            pl.BlockSpec((bm, Cmid), lambda i: (i, 0)),
            pl.BlockSpec((Cin, Cmid), lambda i: (0, 0)),
            pl.BlockSpec((1, Cmid), lambda i: (0, 0)),
            pl.BlockSpec((Cmid, Cout), lambda i: (0, 0)),
        ],
        out_specs=pl.BlockSpec((bm, Cout), lambda i: (i, 0)),
        compiler_params=pltpu.CompilerParams(
            dimension_semantics=("parallel",),
            vmem_limit_bytes=64 * 1024 * 1024,
        ),
        cost_estimate=pl.CostEstimate(
            flops=flops, transcendentals=0, bytes_accessed=bytes_accessed),
    )(x, s, w1t, b, w2t)

    return out.reshape(H, W, N, Cout).transpose(2, 3, 0, 1)


# speedup vs baseline: 6.4434x; 1.3081x over previous
import jax
import jax.numpy as jnp
from jax.experimental import pallas as pl
from jax.experimental.pallas import tpu as pltpu


def _fused_bottleneck(x_ref, s_ref, w1_ref, b_ref, w2_ref, o_ref):
    xb = x_ref[...].astype(jnp.bfloat16)
    h = jax.lax.dot_general(xb, w1_ref[...], (((1,), (1,)), ((), ())),
                            preferred_element_type=jnp.float32)
    h = jnp.maximum(h + b_ref[...] + s_ref[...], 0.0)
    o_ref[...] = jax.lax.dot_general(h.astype(jnp.bfloat16), w2_ref[...],
                                     (((1,), (1,)), ((), ())),
                                     preferred_element_type=jnp.float32)


def kernel(x356, x350, w1, gamma, beta, run_mean, run_var, w2, eps=1e-5):
    N, Cin, H, W = x356.shape
    Cmid = w1.shape[0]
    Cout = w2.shape[0]
    M = N * H * W

    x = x356.transpose(2, 3, 0, 1).reshape(M, Cin)
    s = x350.transpose(2, 3, 0, 1).reshape(M, Cmid)

    inv = gamma * jax.lax.rsqrt(run_var + eps)
    b = (beta - run_mean * inv)[None, :]
    w1t = (w1.reshape(Cmid, Cin) * inv[:, None]).astype(jnp.bfloat16)
    w2t = w2.reshape(Cout, Cmid).astype(jnp.bfloat16)

    bm = 784
    grid = (M // bm,)

    flops = 2 * M * Cmid * (Cin + Cout)
    bytes_accessed = (x.size + s.size + M * Cout) * 4 + (
        w1t.size + w2t.size) * 2 + b.size * 4

    out = pl.pallas_call(
        _fused_bottleneck,
        out_shape=jax.ShapeDtypeStruct((M, Cout), jnp.float32),
        grid=grid,
        in_specs=[
            pl.BlockSpec((bm, Cin), lambda i: (i, 0)),
            pl.BlockSpec((bm, Cmid), lambda i: (i, 0)),
            pl.BlockSpec((Cmid, Cin), lambda i: (0, 0)),
            pl.BlockSpec((1, Cmid), lambda i: (0, 0)),
            pl.BlockSpec((Cout, Cmid), lambda i: (0, 0)),
        ],
        out_specs=pl.BlockSpec((bm, Cout), lambda i: (i, 0)),
        compiler_params=pltpu.CompilerParams(
            dimension_semantics=("parallel",),
            vmem_limit_bytes=64 * 1024 * 1024,
        ),
        cost_estimate=pl.CostEstimate(
            flops=flops, transcendentals=0, bytes_accessed=bytes_accessed),
    )(x, s, w1t, b, w2t)

    return out.reshape(H, W, N, Cout).transpose(2, 3, 0, 1)
